# baseline (speedup 1.0000x reference)
import functools

import jax
import jax.numpy as jnp
from jax import lax
from jax.experimental import pallas as pl
from jax.experimental.pallas import tpu as pltpu
from jax.experimental.pallas import tpu_sc as plsc

NC, NS = 2, 16
NW = NC * NS
CHUNK = 128
SUB = 8
GROUP = SUB * CHUNK


def _sc_gather(ids4, emb):
    _, G, _, _ = ids4.shape
    D = emb.shape[1]

    @functools.partial(
        pl.kernel,
        mesh=plsc.VectorSubcoreMesh(core_axis_name="c", subcore_axis_name="s"),
        out_type=jax.ShapeDtypeStruct((NW * G * GROUP, D), jnp.float32),
        scratch_types=[
            pltpu.VMEM((SUB, CHUNK), jnp.int32),
            pltpu.VMEM((GROUP, D), jnp.float32),
            pltpu.SemaphoreType.DMA,
        ],
        compiler_params=pltpu.CompilerParams(use_tc_tiling_on_sc=False),
    )
    def k(ids_hbm, emb_hbm, out_hbm, idx_v, rows_v, sem):
        wid = lax.axis_index("s") * NC + lax.axis_index("c")

        def body(g, carry):
            pltpu.sync_copy(ids_hbm.at[wid, g], idx_v)
            cps = [
                pltpu.async_copy(
                    emb_hbm.at[idx_v.at[j]],
                    rows_v.at[pl.ds(j * CHUNK, CHUNK)],
                    sem,
                )
                for j in range(SUB)
            ]
            for cp in cps:
                cp.wait()
            pltpu.sync_copy(
                rows_v, out_hbm.at[pl.ds((wid * G + g) * GROUP, GROUP)]
            )
            return carry

        lax.fori_loop(0, G, body, 0)

    return k(ids4, emb)


def _tc_linear(x, Wt, b2, Bt, L, D):
    SB = 32
    RB = SB * L

    def body(x_ref, w_ref, b_ref, o_ref):
        z = (
            jnp.dot(x_ref[...], w_ref[...], preferred_element_type=jnp.float32)
            + b_ref[...]
        )
        o_ref[...] = z.reshape(SB, L, D)

    return pl.pallas_call(
        body,
        grid=(Bt // SB,),
        in_specs=[
            pl.BlockSpec((RB, D), lambda i: (i, 0)),
            pl.BlockSpec((D, D), lambda i: (0, 0)),
            pl.BlockSpec((1, D), lambda i: (0, 0)),
        ],
        out_specs=pl.BlockSpec((SB, L, D), lambda i: (i, 0, 0)),
        out_shape=jax.ShapeDtypeStruct((Bt, L, D), jnp.float32),
    )(x, Wt, b2)


def kernel(input_ids, emb, W, b):
    Bt, L = input_ids.shape
    V, D = emb.shape
    N = Bt * L
    G = N // (NW * GROUP)
    ids4 = input_ids.astype(jnp.int32).reshape(NW, G, SUB, CHUNK)
    x = _sc_gather(ids4, emb)
    return _tc_linear(x, W.T, b.reshape(1, D), Bt, L, D)


# speedup vs baseline: 1.1745x; 1.1745x over previous
import functools

import jax
import jax.numpy as jnp
from jax import lax
from jax.experimental import pallas as pl
from jax.experimental.pallas import tpu as pltpu
from jax.experimental.pallas import tpu_sc as plsc

NC, NS = 2, 16
NW = NC * NS
CHUNK = 128
PH = 4
GROUP = PH * CHUNK


def _sc_gather_packed(ids4, emb):
    _, G, _, _ = ids4.shape
    D = emb.shape[1]

    @functools.partial(
        pl.kernel,
        mesh=plsc.VectorSubcoreMesh(core_axis_name="c", subcore_axis_name="s"),
        out_type=jax.ShapeDtypeStruct((NW * G * CHUNK, PH * D), jnp.float32),
        scratch_types=[
            pltpu.VMEM((PH, CHUNK), jnp.int32),
            pltpu.VMEM((PH, CHUNK, D), jnp.float32),
            pltpu.SemaphoreType.DMA,
            pltpu.SemaphoreType.DMA,
        ],
        compiler_params=pltpu.CompilerParams(use_tc_tiling_on_sc=False),
    )
    def k(ids_hbm, emb_hbm, out_hbm, idx_v, rows_v, sem, sem2):
        wid = lax.axis_index("s") * NC + lax.axis_index("c")

        def body(g, carry):
            pltpu.sync_copy(ids_hbm.at[wid, g], idx_v)
            cps = [
                pltpu.async_copy(
                    emb_hbm.at[idx_v.at[k_]],
                    rows_v.at[k_],
                    sem,
                )
                for k_ in range(PH)
            ]
            for cp in cps:
                cp.wait()
            base = (wid * G + g) * CHUNK
            outs = [
                pltpu.async_copy(
                    rows_v.at[k_],
                    out_hbm.at[pl.ds(base, CHUNK), pl.ds(k_ * D, D)],
                    sem2,
                )
                for k_ in range(PH)
            ]
            for cp in outs:
                cp.wait()
            return carry

        lax.fori_loop(0, G, body, 0)

    return k(ids4, emb)


def _tc_linear(xp, w4, b4):
    M = xp.shape[0]
    BM = 2048

    def body(x_ref, w_ref, b_ref, o_ref):
        o_ref[...] = (
            jnp.dot(x_ref[...], w_ref[...], preferred_element_type=jnp.float32)
            + b_ref[...]
        )

    return pl.pallas_call(
        body,
        grid=(M // BM,),
        in_specs=[
            pl.BlockSpec((BM, 128), lambda i: (i, 0)),
            pl.BlockSpec((128, 128), lambda i: (0, 0)),
            pl.BlockSpec((1, 128), lambda i: (0, 0)),
        ],
        out_specs=pl.BlockSpec((BM, 128), lambda i: (i, 0)),
        out_shape=jax.ShapeDtypeStruct((M, 128), jnp.float32),
    )(xp, w4, b4)


def kernel(input_ids, emb, W, b):
    Bt, L = input_ids.shape
    V, D = emb.shape
    N = Bt * L
    G = N // (NW * GROUP)
    ids4 = (
        input_ids.astype(jnp.int32)
        .reshape(NW, G, CHUNK, PH)
        .transpose(0, 1, 3, 2)
    )
    xp = _sc_gather_packed(ids4, emb)
    w4 = jnp.kron(jnp.eye(4, dtype=W.dtype), W.T)
    b4 = jnp.tile(b, 4).reshape(1, 4 * D)
    y = _tc_linear(xp, w4, b4)
    return y.reshape(Bt, L, D)
